# Initial kernel scaffold; baseline (speedup 1.0000x reference)
import jax
import jax.numpy as jnp
from jax.experimental import pallas as pl


def kernel(input_ids, table):
    raise NotImplementedError("write your pallas kernel here")


# speedup vs baseline: 1.8748x; 1.8748x over previous
import functools

import jax
import jax.numpy as jnp
from jax import lax
from jax.experimental import pallas as pl
from jax.experimental.pallas import tpu as pltpu
from jax.experimental.pallas import tpu_sc as plsc

HIDDEN = 64
NC = 2
NS = 16
NW = NC * NS

CH = 128
NBUF = 8


@functools.cache
def _make_sc_lookup(n_total):
    per_w = n_total // NW
    nch = per_w // CH
    ngrp = nch // NBUF
    mesh = plsc.VectorSubcoreMesh(core_axis_name="c", subcore_axis_name="s")

    @functools.partial(
        pl.kernel,
        mesh=mesh,
        out_type=jax.ShapeDtypeStruct((n_total, HIDDEN), jnp.float32),
        scratch_types=[
            pltpu.VMEM((nch, CH), jnp.int32),
            pltpu.VMEM((NBUF, CH, HIDDEN), jnp.float32),
            pltpu.SemaphoreType.DMA,
            pltpu.SemaphoreType.DMA,
        ],
        compiler_params=pltpu.CompilerParams(use_tc_tiling_on_sc=False),
    )
    def lookup(idx_hbm, table_hbm, out_hbm, idx_v, rows_v, gsem, osem):
        wid = lax.axis_index("s") * NC + lax.axis_index("c")
        base = wid * per_w
        pltpu.sync_copy(idx_hbm.at[wid], idx_v)

        def group(g, carry):
            row0 = base + g * (NBUF * CH)
            gathers = [
                pltpu.async_copy(
                    table_hbm.at[idx_v.at[g * NBUF + b]], rows_v.at[b], gsem)
                for b in range(NBUF)
            ]
            writes = []
            for b in range(NBUF):
                gathers[b].wait()
                writes.append(pltpu.async_copy(
                    rows_v.at[b], out_hbm.at[pl.ds(row0 + b * CH, CH)], osem))
            for w in writes:
                w.wait()
            return carry

        lax.fori_loop(0, ngrp, group, 0)

    return lookup


def kernel(input_ids, table):
    batch, hist = input_ids.shape
    n_total = batch * hist
    idx = input_ids.astype(jnp.int32).reshape(NW, n_total // (NW * CH), CH)
    out = _make_sc_lookup(n_total)(idx, table)
    return out.reshape(batch, hist, HIDDEN)
